# baseline (speedup 1.0000x reference)
import functools

import jax
import jax.numpy as jnp
from jax.experimental import pallas as pl
from jax.experimental.pallas import tpu as pltpu


_SPLIT = 4


def _mlp_body(*refs):
    x_refs = refs[:_SPLIT]
    w1t_ref, b1_ref, w2t_ref, b2_ref, o_ref = refs[_SPLIT:]
    w1t = w1t_ref[...]
    w2t = w2t_ref[...]
    b1 = b1_ref[...]
    b2 = b2_ref[...]
    sub = x_refs[0].shape[1]
    for k in range(_SPLIT):
        x = x_refs[k][...]
        h = jnp.dot(w1t, x, preferred_element_type=jnp.float32)
        h = jnp.maximum(h + b1, 0.0)
        y = jnp.dot(w2t, h, preferred_element_type=jnp.float32)
        o_ref[:, k * sub:(k + 1) * sub] = jnp.maximum(y + b2, 0.0)


@functools.partial(jax.jit, static_argnames=("block_cols",))
def _run(xt, w1t, b1c, w2t, b2c, block_cols):
    d, e = xt.shape
    grid = e // block_cols
    sub = block_cols // _SPLIT

    def _make_in_spec(k):
        return pl.BlockSpec((d, sub), lambda i, _k=k: (0, _SPLIT * i + _k))

    return pl.pallas_call(
        _mlp_body,
        grid=(grid,),
        in_specs=[_make_in_spec(k) for k in range(_SPLIT)] + [
            pl.BlockSpec((d, d), lambda i: (0, 0)),
            pl.BlockSpec((d, 1), lambda i: (0, 0)),
            pl.BlockSpec((d, d), lambda i: (0, 0)),
            pl.BlockSpec((d, 1), lambda i: (0, 0)),
        ],
        out_specs=pl.BlockSpec((d, block_cols), lambda i: (0, i)),
        out_shape=jax.ShapeDtypeStruct((d, e), jnp.float32),
        compiler_params=pltpu.CompilerParams(
            dimension_semantics=("arbitrary",),
        ),
    )(xt, xt, xt, xt, w1t, b1c, w2t, b2c)


def kernel(edge_attr, W1, b1, W2, b2):
    e, d = edge_attr.shape
    xt = edge_attr.T
    w1t = W1.astype(jnp.float32).T
    w2t = W2.astype(jnp.float32).T
    b1c = b1.astype(jnp.float32).reshape(d, 1)
    b2c = b2.astype(jnp.float32).reshape(d, 1)
    block_cols = next(bc for bc in (128000, 64000, 32000, 16000, 2048, 512)
                      if e % bc == 0 and (bc // _SPLIT) % 128 == 0)
    out_t = _run(xt, w1t, b1c, w2t, b2c, block_cols=block_cols)
    return out_t.T


# speedup vs baseline: 1.0052x; 1.0052x over previous
import functools

import jax
import jax.numpy as jnp
from jax.experimental import pallas as pl
from jax.experimental.pallas import tpu as pltpu


_SPLIT = 2


def _mlp_body(*refs):
    x_refs = refs[:_SPLIT]
    w1t_ref, b1_ref, w2t_ref, b2_ref, o_ref = refs[_SPLIT:]
    w1t = w1t_ref[...]
    w2t = w2t_ref[...]
    b1 = b1_ref[...]
    b2 = b2_ref[...]
    sub = x_refs[0].shape[1]
    for k in range(_SPLIT):
        x = x_refs[k][...]
        h = jnp.dot(w1t, x, preferred_element_type=jnp.float32)
        h = jnp.maximum(h + b1, 0.0)
        y = jnp.dot(w2t, h, preferred_element_type=jnp.float32)
        o_ref[:, k * sub:(k + 1) * sub] = jnp.maximum(y + b2, 0.0)


@functools.partial(jax.jit, static_argnames=("block_cols",))
def _run(xt, w1t, b1c, w2t, b2c, block_cols):
    d, e = xt.shape
    grid = e // block_cols
    sub = block_cols // _SPLIT

    def _make_in_spec(k):
        return pl.BlockSpec((d, sub), lambda i, _k=k: (0, _SPLIT * i + _k))

    return pl.pallas_call(
        _mlp_body,
        grid=(grid,),
        in_specs=[_make_in_spec(k) for k in range(_SPLIT)] + [
            pl.BlockSpec((d, d), lambda i: (0, 0)),
            pl.BlockSpec((d, 1), lambda i: (0, 0)),
            pl.BlockSpec((d, d), lambda i: (0, 0)),
            pl.BlockSpec((d, 1), lambda i: (0, 0)),
        ],
        out_specs=pl.BlockSpec((d, block_cols), lambda i: (0, i)),
        out_shape=jax.ShapeDtypeStruct((d, e), jnp.float32),
        compiler_params=pltpu.CompilerParams(
            dimension_semantics=("arbitrary",),
        ),
    )(xt, xt, w1t, b1c, w2t, b2c)


def kernel(edge_attr, W1, b1, W2, b2):
    e, d = edge_attr.shape
    xt = edge_attr.T
    w1t = W1.astype(jnp.float32).T
    w2t = W2.astype(jnp.float32).T
    b1c = b1.astype(jnp.float32).reshape(d, 1)
    b2c = b2.astype(jnp.float32).reshape(d, 1)
    block_cols = next(bc for bc in (160000, 128000, 64000, 32000, 2048, 512)
                      if e % bc == 0 and (bc // _SPLIT) % 128 == 0)
    out_t = _run(xt, w1t, b1c, w2t, b2c, block_cols=block_cols)
    return out_t.T


# speedup vs baseline: 1.0056x; 1.0004x over previous
import functools

import jax
import jax.numpy as jnp
from jax.experimental import pallas as pl
from jax.experimental.pallas import tpu as pltpu


def _mlp_body(x_ref, w1t_ref, b1_ref, w2t_ref, b2_ref, o_ref):
    x = x_ref[...]
    h = jnp.dot(w1t_ref[...], x, preferred_element_type=jnp.float32)
    h = jnp.maximum(h + b1_ref[...], 0.0)
    y = jnp.dot(w2t_ref[...], h, preferred_element_type=jnp.float32)
    o_ref[...] = jnp.maximum(y + b2_ref[...], 0.0)


@functools.partial(jax.jit, static_argnames=("block_cols",))
def _run(xt, w1t, b1c, w2t, b2c, block_cols):
    d, e = xt.shape
    grid = e // block_cols
    return pl.pallas_call(
        _mlp_body,
        grid=(grid,),
        in_specs=[
            pl.BlockSpec((d, block_cols), lambda i: (0, i)),
            pl.BlockSpec((d, d), lambda i: (0, 0)),
            pl.BlockSpec((d, 1), lambda i: (0, 0)),
            pl.BlockSpec((d, d), lambda i: (0, 0)),
            pl.BlockSpec((d, 1), lambda i: (0, 0)),
        ],
        out_specs=pl.BlockSpec((d, block_cols), lambda i: (0, i)),
        out_shape=jax.ShapeDtypeStruct((d, e), jnp.float32),
        compiler_params=pltpu.CompilerParams(
            dimension_semantics=("parallel",),
        ),
    )(xt, w1t, b1c, w2t, b2c)


def kernel(edge_attr, W1, b1, W2, b2):
    e, d = edge_attr.shape
    xt = edge_attr.T
    w1t = W1.astype(jnp.float32).T
    w2t = W2.astype(jnp.float32).T
    b1c = b1.astype(jnp.float32).reshape(d, 1)
    b2c = b2.astype(jnp.float32).reshape(d, 1)
    block_cols = next(bc for bc in (160000, 64000, 32000, 16000, 8000, 4000,
                                    2000, 1000, 128)
                      if e % bc == 0)
    out_t = _run(xt, w1t, b1c, w2t, b2c, block_cols=block_cols)
    return out_t.T
